# baseline (speedup 1.0000x reference)
import functools

import jax
import jax.numpy as jnp
from jax import lax
from jax.experimental import pallas as pl
from jax.experimental.pallas import tpu as pltpu
from jax.experimental.pallas import tpu_sc as plsc

_NC = 2
_NS = 16
_NW = _NC * _NS


def _gather_rows(tabT_flat, idx, V, D):
    B = idx.shape[0]
    tiles_per_row = _NW // D
    n = B // tiles_per_row
    mesh = plsc.VectorSubcoreMesh(core_axis_name="c", subcore_axis_name="s")

    @functools.partial(
        pl.kernel,
        mesh=mesh,
        out_type=jax.ShapeDtypeStruct((D * B,), jnp.float32),
        scratch_types=[
            pltpu.VMEM((n,), jnp.int32),
            pltpu.VMEM((n,), jnp.float32),
            pltpu.SemaphoreType.DMA,
        ],
        compiler_params=pltpu.CompilerParams(
            use_tc_tiling_on_sc=False, needs_layout_passes=False
        ),
    )
    def gather_kernel(tab_hbm, idx_hbm, out_hbm, idx_v, gath_v, sem):
        wid = lax.axis_index("s") * _NC + lax.axis_index("c")
        k = wid // tiles_per_row
        part = wid % tiles_per_row
        pltpu.sync_copy(idx_hbm.at[pl.ds(part * n, n)], idx_v)
        for c in range(n // 16):
            sl = pl.ds(c * 16, 16)
            idx_v[sl] = idx_v[sl] + k * V
        pltpu.async_copy(tab_hbm.at[idx_v], gath_v, sem).wait()
        pltpu.sync_copy(gath_v, out_hbm.at[pl.ds(k * B + part * n, n)])

    return gather_kernel(tabT_flat, idx).reshape(D, B)


def _matmul_body(wt_ref, ht_ref, b_ref, o_ref):
    ot = lax.dot_general(
        wt_ref[...],
        ht_ref[...],
        dimension_numbers=(((0,), (0,)), ((), ())),
        preferred_element_type=jnp.float32,
    )
    bias = b_ref[...]
    o_ref[...] = ot + lax.transpose(bias, (1, 0))


def _project_t(ht, Wt, b2d, blk):
    D, B = ht.shape
    V = Wt.shape[1]
    return pl.pallas_call(
        _matmul_body,
        grid=(pl.cdiv(V, blk),),
        in_specs=[
            pl.BlockSpec((D, blk), lambda j: (0, j)),
            pl.BlockSpec((D, B), lambda j: (0, 0)),
            pl.BlockSpec((1, blk), lambda j: (0, j)),
        ],
        out_specs=pl.BlockSpec((blk, B), lambda j: (j, 0)),
        out_shape=jax.ShapeDtypeStruct((V, B), jnp.float32),
    )(Wt, ht, b2d)


def kernel(x, emb_table, W, b):
    x = x.astype(jnp.int32)
    V, D = emb_table.shape
    ht = _gather_rows(emb_table.T.reshape(-1), x, V, D)
    ot = _project_t(ht, W.T, b.reshape(1, -1), blk=1024)
    return ot.T


# speedup vs baseline: 1.1308x; 1.1308x over previous
import functools

import jax
import jax.numpy as jnp
from jax import lax
from jax.experimental import pallas as pl
from jax.experimental.pallas import tpu as pltpu
from jax.experimental.pallas import tpu_sc as plsc

_NC = 2
_NS = 16
_NW = _NC * _NS


def _gather_rows(tabT_flat, idx, V, D):
    B = idx.shape[0]
    tiles_per_row = _NW // D
    n = B // tiles_per_row
    mesh = plsc.VectorSubcoreMesh(core_axis_name="c", subcore_axis_name="s")

    @functools.partial(
        pl.kernel,
        mesh=mesh,
        out_type=jax.ShapeDtypeStruct((D * B,), jnp.float32),
        scratch_types=[
            pltpu.VMEM((n,), jnp.int32),
            pltpu.VMEM((n,), jnp.float32),
            pltpu.SemaphoreType.DMA,
        ],
        compiler_params=pltpu.CompilerParams(
            use_tc_tiling_on_sc=False, needs_layout_passes=False
        ),
    )
    def gather_kernel(tab_hbm, idx_hbm, out_hbm, idx_v, gath_v, sem):
        wid = lax.axis_index("s") * _NC + lax.axis_index("c")
        k = wid // tiles_per_row
        part = wid % tiles_per_row
        pltpu.sync_copy(idx_hbm.at[pl.ds(part * n, n)], idx_v)
        for c in range(n // 16):
            sl = pl.ds(c * 16, 16)
            idx_v[sl] = idx_v[sl] + k * V
        pltpu.async_copy(tab_hbm.at[idx_v], gath_v, sem).wait()
        pltpu.sync_copy(gath_v, out_hbm.at[pl.ds(k * B + part * n, n)])

    return gather_kernel(tabT_flat, idx).reshape(D, B)


def _matmul_body(wt_ref, ht_ref, b_ref, o_ref):
    ot = lax.dot_general(
        wt_ref[...],
        ht_ref[...],
        dimension_numbers=(((0,), (0,)), ((), ())),
        preferred_element_type=jnp.float32,
    )
    bias = b_ref[...]
    o_ref[...] = ot + lax.transpose(bias, (1, 0))


def _project_t(ht, Wt, b2d, blk):
    D, B = ht.shape
    V = Wt.shape[1]
    return pl.pallas_call(
        _matmul_body,
        grid=(pl.cdiv(V, blk),),
        in_specs=[
            pl.BlockSpec((D, blk), lambda j: (0, j)),
            pl.BlockSpec((D, B), lambda j: (0, 0)),
            pl.BlockSpec((1, blk), lambda j: (0, j)),
        ],
        out_specs=pl.BlockSpec((blk, B), lambda j: (j, 0)),
        out_shape=jax.ShapeDtypeStruct((V, B), jnp.float32),
    )(Wt, ht, b2d)


def kernel(x, emb_table, W, b):
    x = x.astype(jnp.int32)
    V, D = emb_table.shape
    ht = _gather_rows(emb_table.T.reshape(-1), x, V, D)
    ot = _project_t(ht, W.T, b.reshape(1, -1), blk=2560)
    return ot.T


# speedup vs baseline: 1.1387x; 1.0069x over previous
import functools

import jax
import jax.numpy as jnp
from jax import lax
from jax.experimental import pallas as pl
from jax.experimental.pallas import tpu as pltpu
from jax.experimental.pallas import tpu_sc as plsc

_NC = 2
_NS = 16
_NW = _NC * _NS


def _gather_rows(tabT_flat, idx, V, D):
    B = idx.shape[0]
    tiles_per_row = _NW // D
    n = B // tiles_per_row
    mesh = plsc.VectorSubcoreMesh(core_axis_name="c", subcore_axis_name="s")

    @functools.partial(
        pl.kernel,
        mesh=mesh,
        out_type=jax.ShapeDtypeStruct((D * B,), jnp.float32),
        scratch_types=[
            pltpu.VMEM((n,), jnp.int32),
            pltpu.VMEM((n,), jnp.float32),
            pltpu.SemaphoreType.DMA,
        ],
        compiler_params=pltpu.CompilerParams(
            use_tc_tiling_on_sc=False,
            needs_layout_passes=False,
            skip_device_barrier=True,
        ),
    )
    def gather_kernel(tab_hbm, idx_hbm, out_hbm, idx_v, gath_v, sem):
        wid = lax.axis_index("s") * _NC + lax.axis_index("c")
        k = wid // tiles_per_row
        part = wid % tiles_per_row
        pltpu.sync_copy(idx_hbm.at[pl.ds(part * n, n)], idx_v)
        for c in range(n // 16):
            sl = pl.ds(c * 16, 16)
            idx_v[sl] = idx_v[sl] + k * V
        pltpu.async_copy(tab_hbm.at[idx_v], gath_v, sem).wait()
        pltpu.sync_copy(gath_v, out_hbm.at[pl.ds(k * B + part * n, n)])

    return gather_kernel(tabT_flat, idx).reshape(D, B)


def _matmul_body(wt_ref, ht_ref, b_ref, o_ref):
    ot = lax.dot_general(
        wt_ref[...],
        ht_ref[...],
        dimension_numbers=(((0,), (0,)), ((), ())),
        preferred_element_type=jnp.float32,
    )
    bias = b_ref[...]
    o_ref[...] = ot + lax.transpose(bias, (1, 0))


def _project_t(ht, Wt, b2d, blk):
    D, B = ht.shape
    V = Wt.shape[1]
    return pl.pallas_call(
        _matmul_body,
        grid=(pl.cdiv(V, blk),),
        in_specs=[
            pl.BlockSpec((D, blk), lambda j: (0, j)),
            pl.BlockSpec((D, B), lambda j: (0, 0)),
            pl.BlockSpec((1, blk), lambda j: (0, j)),
        ],
        out_specs=pl.BlockSpec((blk, B), lambda j: (j, 0)),
        out_shape=jax.ShapeDtypeStruct((V, B), jnp.float32),
    )(Wt, ht, b2d)


def kernel(x, emb_table, W, b):
    x = x.astype(jnp.int32)
    V, D = emb_table.shape
    ht = _gather_rows(emb_table.T.reshape(-1), x, V, D)
    ot = _project_t(ht, W.T, b.reshape(1, -1), blk=2048)
    return ot.T
